# baseline (speedup 1.0000x reference)
import functools

import jax
import jax.numpy as jnp
from jax import lax
from jax.experimental import pallas as pl
from jax.experimental.pallas import tpu as pltpu
from jax.experimental.pallas import tpu_sc as plsc

N_NODES = 10000
N_PAD = 10240
FEAT = 128
BM = 1000

NUM_CORES = 2
NUM_TILES = 16
NWORK = NUM_CORES * NUM_TILES
WAVE_E = 64
NDESC = 4
DESC_E = 16
NWAVES = 160
E_PAD = NWORK * NWAVES * WAVE_E
NSLOT = 4
SUPERW = 8
ROWS_PER_TILE_N = N_PAD // NUM_TILES
ZROWS = WAVE_E


def _sc_gather_body(x_hbm, src_hbm, msg_hbm,
                    table, src_v, s16, rows_v, gsem, wsem):
    c = lax.axis_index("c")
    s = lax.axis_index("s")
    r0 = s * ROWS_PER_TILE_N
    wid = c * NUM_TILES + s

    pltpu.sync_copy(x_hbm.at[pl.ds(r0, ROWS_PER_TILE_N), :],
                    table.at[pl.ds(r0, ROWS_PER_TILE_N), :])

    def stage(sc):
        sb = lax.rem(sc, 2)
        pltpu.sync_copy(src_hbm.at[c, s, pl.ds(sc * SUPERW, SUPERW), :],
                        src_v.at[sb])

    def fill_idx(w, slot):
        sb = lax.rem(lax.div(w, SUPERW), 2)
        wl = lax.rem(w, SUPERW)
        for i in range(NDESC):
            s16[slot, i, :] = src_v[sb, wl, pl.ds(i * DESC_E, DESC_E)]

    def fire_gathers(slot):
        for i in range(NDESC):
            pltpu.async_copy(table.at[s16.at[slot, i]],
                             rows_v.at[slot, pl.ds(i * DESC_E, DESC_E), :],
                             gsem.at[slot])

    def drain_gathers(slot):
        for i in range(NDESC):
            pltpu.make_async_copy(
                table.at[s16.at[0, 0]],
                rows_v.at[slot, pl.ds(i * DESC_E, DESC_E), :],
                gsem.at[slot]).wait()

    def fire_write(w, slot):
        goff = (wid * NWAVES + w) * WAVE_E
        pltpu.async_copy(rows_v.at[slot], msg_hbm.at[pl.ds(goff, WAVE_E), :],
                         wsem.at[slot])

    def drain_write(slot):
        pltpu.make_async_copy(rows_v.at[slot],
                              msg_hbm.at[pl.ds(0, WAVE_E), :],
                              wsem.at[slot]).wait()

    stage(0)
    fill_idx(0, 0)
    fill_idx(1, 1)
    plsc.subcore_barrier()
    fire_gathers(0)
    fire_gathers(1)

    def _body(w, _):
        slot = lax.rem(w, NSLOT)
        slot2 = lax.rem(w + 2, NSLOT)

        @pl.when(w + 2 < NWAVES)
        def _():
            @pl.when(lax.rem(w + 2, SUPERW) == 0)
            def _():
                stage(lax.div(w + 2, SUPERW))

            @pl.when(w >= 2)
            def _():
                drain_write(slot2)

            fill_idx(w + 2, slot2)
            fire_gathers(slot2)

        drain_gathers(slot)
        fire_write(w, slot)
        return 0
    lax.fori_loop(0, NWAVES, _body, 0)

    for k in range(NSLOT):
        drain_write(k)


_sc_gather = functools.partial(
    pl.kernel,
    out_type=jax.ShapeDtypeStruct((E_PAD, FEAT), jnp.float32),
    mesh=plsc.VectorSubcoreMesh(core_axis_name="c", subcore_axis_name="s"),
    scratch_types=[
        pltpu.MemorySpace.VMEM_SHARED((N_PAD, FEAT), jnp.float32),
        pltpu.MemorySpace.VMEM((2, SUPERW, WAVE_E), jnp.int32),
        pltpu.MemorySpace.VMEM((NSLOT, NDESC, DESC_E), jnp.int32),
        pltpu.MemorySpace.VMEM((NSLOT, WAVE_E, FEAT), jnp.float32),
        pltpu.SemaphoreType.DMA((NSLOT,)),
        pltpu.SemaphoreType.DMA((NSLOT,)),
    ],
)(_sc_gather_body)


def _sc_scatter_body(msg_hbm, dst_hbm, ew_hbm, out_hbm,
                     acc, dst_v, ew_v, d16, rows_v, rsem, ssem):
    c = lax.axis_index("c")
    s = lax.axis_index("s")
    r0 = s * ROWS_PER_TILE_N
    wid = c * NUM_TILES + s

    def _zrow(i, _):
        for q in range(FEAT // 16):
            rows_v[0, i, pl.ds(q * 16, 16)] = jnp.zeros((16,), jnp.float32)
        return 0
    lax.fori_loop(0, ZROWS, _zrow, 0)
    for k in range(ROWS_PER_TILE_N // ZROWS):
        pltpu.sync_copy(rows_v.at[0], acc.at[pl.ds(r0 + k * ZROWS, ZROWS), :])

    def stage(sc):
        sb = lax.rem(sc, 2)
        pltpu.sync_copy(dst_hbm.at[c, s, pl.ds(sc * SUPERW, SUPERW), :],
                        dst_v.at[sb])
        pltpu.sync_copy(ew_hbm.at[c, s, pl.ds(sc * SUPERW, SUPERW), :],
                        ew_v.at[sb])

    def fill_idx(w, slot):
        sb = lax.rem(lax.div(w, SUPERW), 2)
        wl = lax.rem(w, SUPERW)
        for i in range(NDESC):
            d16[slot, i, :] = dst_v[sb, wl, pl.ds(i * DESC_E, DESC_E)]

    def fire_read(w, slot):
        goff = (wid * NWAVES + w) * WAVE_E
        pltpu.async_copy(msg_hbm.at[pl.ds(goff, WAVE_E), :], rows_v.at[slot],
                         rsem.at[slot])

    def drain_read(slot):
        pltpu.make_async_copy(msg_hbm.at[pl.ds(0, WAVE_E), :],
                              rows_v.at[slot], rsem.at[slot]).wait()

    def fire_scatters(slot):
        for i in range(NDESC):
            pltpu.async_copy(rows_v.at[slot, pl.ds(i * DESC_E, DESC_E), :],
                             acc.at[d16.at[slot, i]], ssem.at[slot], add=True)

    def drain_scatters(slot):
        for i in range(NDESC):
            pltpu.make_async_copy(
                rows_v.at[slot, pl.ds(i * DESC_E, DESC_E), :],
                acc.at[d16.at[0, 0]], ssem.at[slot]).wait()

    def mul(w, slot):
        sb = lax.rem(lax.div(w, SUPERW), 2)
        wl = lax.rem(w, SUPERW)
        def _desc(d, _):
            e16 = ew_v[sb, wl, pl.ds(d * DESC_E, DESC_E)]
            for l in range(DESC_E):
                e = lax.gather(
                    e16, jnp.full((16, 1), l, jnp.int32),
                    dimension_numbers=lax.GatherDimensionNumbers(
                        offset_dims=(), collapsed_slice_dims=(0,),
                        start_index_map=(0,)),
                    slice_sizes=(1,),
                    mode=lax.GatherScatterMode.PROMISE_IN_BOUNDS)
                k = d * DESC_E + l
                for q in range(FEAT // 16):
                    sl = pl.ds(q * 16, 16)
                    rows_v[slot, k, sl] = rows_v[slot, k, sl] * e
            return 0
        lax.fori_loop(0, NDESC, _desc, 0)

    stage(0)
    fill_idx(0, 0)
    fill_idx(1, 1)
    plsc.subcore_barrier()
    fire_read(0, 0)
    fire_read(1, 1)

    def _body(w, _):
        slot = lax.rem(w, NSLOT)
        slot2 = lax.rem(w + 2, NSLOT)

        @pl.when(w + 2 < NWAVES)
        def _():
            @pl.when(lax.rem(w + 2, SUPERW) == 0)
            def _():
                stage(lax.div(w + 2, SUPERW))

            @pl.when(w >= 2)
            def _():
                drain_scatters(slot2)

            fill_idx(w + 2, slot2)
            fire_read(w + 2, slot2)

        drain_read(slot)
        mul(w, slot)
        fire_scatters(slot)
        return 0
    lax.fori_loop(0, NWAVES, _body, 0)

    for k in range(NSLOT):
        drain_scatters(k)

    plsc.subcore_barrier()
    pltpu.sync_copy(acc.at[pl.ds(r0, ROWS_PER_TILE_N), :],
                    out_hbm.at[c, pl.ds(r0, ROWS_PER_TILE_N), :])


_sc_scatter = functools.partial(
    pl.kernel,
    out_type=jax.ShapeDtypeStruct((NUM_CORES, N_PAD, FEAT), jnp.float32),
    mesh=plsc.VectorSubcoreMesh(core_axis_name="c", subcore_axis_name="s"),
    scratch_types=[
        pltpu.MemorySpace.VMEM_SHARED((N_PAD, FEAT), jnp.float32),
        pltpu.MemorySpace.VMEM((2, SUPERW, WAVE_E), jnp.int32),
        pltpu.MemorySpace.VMEM((2, SUPERW, WAVE_E), jnp.float32),
        pltpu.MemorySpace.VMEM((NSLOT, NDESC, DESC_E), jnp.int32),
        pltpu.MemorySpace.VMEM((NSLOT, WAVE_E, FEAT), jnp.float32),
        pltpu.SemaphoreType.DMA((NSLOT,)),
        pltpu.SemaphoreType.DMA((NSLOT,)),
    ],
)(_sc_scatter_body)


def _layer_body(p_ref, x_ref, wr_ref, wt_ref, b_ref, o_ref):
    agg = p_ref[0] + p_ref[1]
    o = jax.lax.dot_general(agg, wr_ref[...], (((1,), (1,)), ((), ())),
                            preferred_element_type=jnp.float32)
    o += jax.lax.dot_general(x_ref[...], wt_ref[...], (((1,), (1,)), ((), ())),
                             preferred_element_type=jnp.float32)
    o += b_ref[...]
    o_ref[...] = jnp.maximum(o, 0.0)


def _tc_layer(parts, x, W_rel, b_rel, W_root):
    n = x.shape[0]
    return pl.pallas_call(
        _layer_body,
        grid=(n // BM,),
        in_specs=[
            pl.BlockSpec((NUM_CORES, BM, FEAT), lambda i: (0, i, 0)),
            pl.BlockSpec((BM, FEAT), lambda i: (i, 0)),
            pl.BlockSpec((FEAT, FEAT), lambda i: (0, 0)),
            pl.BlockSpec((FEAT, FEAT), lambda i: (0, 0)),
            pl.BlockSpec((1, FEAT), lambda i: (0, 0)),
        ],
        out_specs=pl.BlockSpec((BM, FEAT), lambda i: (i, 0)),
        out_shape=jax.ShapeDtypeStruct((n, FEAT), jnp.float32),
    )(parts, x, W_rel, W_root, b_rel.reshape(1, FEAT))


def _head_body(x1_ref, x2_ref, x3_ref, w_ref, b_ref, o_ref):
    w = w_ref[...]
    l = jax.lax.dot_general(x1_ref[...], w[:, 0:128], (((1,), (1,)), ((), ())),
                            preferred_element_type=jnp.float32)
    l += jax.lax.dot_general(x2_ref[...], w[:, 128:256], (((1,), (1,)), ((), ())),
                             preferred_element_type=jnp.float32)
    l += jax.lax.dot_general(x3_ref[...], w[:, 256:384], (((1,), (1,)), ((), ())),
                             preferred_element_type=jnp.float32)
    l += b_ref[...]
    m = jnp.max(l, axis=-1, keepdims=True)
    lse = jnp.log(jnp.sum(jnp.exp(l - m), axis=-1, keepdims=True))
    o_ref[...] = l - m - lse


def _tc_head(x1, x2, x3, W_lin, b_lin):
    n = x1.shape[0]
    c = W_lin.shape[0]
    return pl.pallas_call(
        _head_body,
        grid=(n // BM,),
        in_specs=[
            pl.BlockSpec((BM, FEAT), lambda i: (i, 0)),
            pl.BlockSpec((BM, FEAT), lambda i: (i, 0)),
            pl.BlockSpec((BM, FEAT), lambda i: (i, 0)),
            pl.BlockSpec((c, 3 * FEAT), lambda i: (0, 0)),
            pl.BlockSpec((1, c), lambda i: (0, 0)),
        ],
        out_specs=pl.BlockSpec((BM, c), lambda i: (i, 0)),
        out_shape=jax.ShapeDtypeStruct((n, c), jnp.float32),
    )(x1, x2, x3, W_lin, b_lin.reshape(1, c))


def kernel(x0, edge_index, edge_weight,
           W_rel1, b_rel1, W_root1,
           W_rel2, b_rel2, W_root2,
           W_rel3, b_rel3, W_root3,
           W_lin, b_lin):
    pad = E_PAD - edge_index.shape[1]
    src = jnp.concatenate([edge_index[0], jnp.zeros((pad,), jnp.int32)])
    dst = jnp.concatenate([edge_index[1], jnp.zeros((pad,), jnp.int32)])
    ew = jnp.concatenate([edge_weight, jnp.zeros((pad,), jnp.float32)])
    eshape = (NUM_CORES, NUM_TILES, NWAVES, WAVE_E)
    src_r = src.reshape(eshape)
    dst_r = dst.reshape(eshape)
    ew_r = ew.reshape(eshape)
    zrows = jnp.zeros((N_PAD - N_NODES, FEAT), jnp.float32)

    def agg(x):
        x_p = jnp.concatenate([x, zrows])
        msg = _sc_gather(x_p, src_r)
        return _sc_scatter(msg, dst_r, ew_r)

    x1 = _tc_layer(agg(x0), x0, W_rel1, b_rel1, W_root1)
    x2 = _tc_layer(agg(x1), x1, W_rel2, b_rel2, W_root2)
    x3 = _tc_layer(agg(x2), x2, W_rel3, b_rel3, W_root3)
    return _tc_head(x1, x2, x3, W_lin, b_lin)
